# Initial kernel scaffold; baseline (speedup 1.0000x reference)
import jax
import jax.numpy as jnp
from jax.experimental import pallas as pl


def kernel(sequence, table, prefix_map, postfix_map):
    raise NotImplementedError("write your pallas kernel here")


# speedup vs baseline: 26.1126x; 26.1126x over previous
import functools

import jax
import jax.numpy as jnp
from jax import lax
from jax.experimental import pallas as pl
from jax.experimental.pallas import tpu as pltpu
from jax.experimental.pallas import tpu_sc as plsc

NC, NS, LANES = 2, 16, 16
NW = NC * NS

V = 100000
D = 64
B = 4096
L = 200
TOK = B * L

C1 = 80
NCH1 = V // C1
ITER1 = (NCH1 + NW - 1) // NW

C2 = 128
PER_W2 = TOK // NW
NCH2 = PER_W2 // C2

_MESH = plsc.VectorSubcoreMesh(
    core_axis_name="c", subcore_axis_name="s", num_cores=NC, num_subcores=NS
)
_PARAMS = pltpu.CompilerParams(use_tc_tiling_on_sc=False)


def _wid():
  return lax.axis_index("s") * NC + lax.axis_index("c")


@functools.partial(
    pl.kernel,
    out_type=jax.ShapeDtypeStruct((V, D), jnp.float32),
    mesh=_MESH,
    compiler_params=_PARAMS,
    scratch_types=[
        pltpu.VMEM((C1,), jnp.int32),
        pltpu.VMEM((C1,), jnp.int32),
        pltpu.VMEM((C1, D), jnp.float32),
        pltpu.VMEM((C1, D), jnp.float32),
        pltpu.VMEM((C1, D), jnp.float32),
        pltpu.SemaphoreType.DMA,
    ],
)
def _build_t2(table, pmap, qmap, t2, pidx, qidx, wrows, prows, qrows, sem):
  wid = _wid()

  def chunk(k, carry):
    i = wid + k * NW

    @pl.when(i < NCH1)
    def _():
      base = i * C1
      pltpu.sync_copy(pmap.at[pl.ds(base, C1)], pidx)
      pltpu.sync_copy(qmap.at[pl.ds(base, C1)], qidx)
      cw = pltpu.async_copy(table.at[pl.ds(base, C1)], wrows, sem)
      cp = pltpu.async_copy(table.at[pidx], prows, sem)
      cq = pltpu.async_copy(table.at[qidx], qrows, sem)
      cw.wait()
      cp.wait()
      cq.wait()

      def row(r, rcarry):
        for j in range(D // LANES):
          s = pl.ds(j * LANES, LANES)
          wrows[r, s] = wrows[r, s] + prows[r, s] + qrows[r, s]
        return rcarry

      lax.fori_loop(0, C1, row, 0)
      pltpu.sync_copy(wrows, t2.at[pl.ds(base, C1)])

    return carry

  lax.fori_loop(0, ITER1, chunk, 0)


@functools.partial(
    pl.kernel,
    out_type=jax.ShapeDtypeStruct((TOK, D), jnp.float32),
    mesh=_MESH,
    compiler_params=_PARAMS,
    scratch_types=[
        pltpu.VMEM((C2,), jnp.int32),
        pltpu.VMEM((C2, D), jnp.float32),
        pltpu.SemaphoreType.DMA,
    ],
)
def _lookup(t2, seq, out, tok, rows, sem):
  wid = _wid()

  def chunk(k, carry):
    base = (wid * NCH2 + k) * C2
    pltpu.sync_copy(seq.at[pl.ds(base, C2)], tok)
    pltpu.async_copy(t2.at[tok], rows, sem).wait()
    pltpu.sync_copy(rows, out.at[pl.ds(base, C2)])
    return carry

  lax.fori_loop(0, NCH2, chunk, 0)


@jax.jit
def kernel(sequence, table, prefix_map, postfix_map):
  t2 = _build_t2(table, prefix_map, postfix_map)
  out = _lookup(t2, sequence.reshape(-1))
  return out.reshape(B, L, D)


# speedup vs baseline: 33.8714x; 1.2971x over previous
import functools

import jax
import jax.numpy as jnp
from jax import lax
from jax.experimental import pallas as pl
from jax.experimental.pallas import tpu as pltpu
from jax.experimental.pallas import tpu_sc as plsc

NC, NS, LANES = 2, 16, 16
NW = NC * NS

V = 100000
D = 64
B = 4096
L = 200

C1 = 80
NCH1 = V // C1
ITER1 = (NCH1 + NW - 1) // NW

BPW = B // NW
LG0 = 128
LG1 = L - LG0

_MESH = plsc.VectorSubcoreMesh(
    core_axis_name="c", subcore_axis_name="s", num_cores=NC, num_subcores=NS
)
_PARAMS = pltpu.CompilerParams(use_tc_tiling_on_sc=False)


def _wid():
  return lax.axis_index("s") * NC + lax.axis_index("c")


@functools.partial(
    pl.kernel,
    out_type=jax.ShapeDtypeStruct((V, D), jnp.float32),
    mesh=_MESH,
    compiler_params=_PARAMS,
    scratch_types=[
        pltpu.VMEM((2, C1), jnp.int32),
        pltpu.VMEM((2, C1), jnp.int32),
        pltpu.VMEM((2, C1, D), jnp.float32),
        pltpu.VMEM((2, C1, D), jnp.float32),
        pltpu.VMEM((2, C1, D), jnp.float32),
        pltpu.SemaphoreType.DMA,
        pltpu.SemaphoreType.DMA,
    ],
)
def _build_t2(table, pmap, qmap, t2, pidx, qidx, wrows, prows, qrows, s0, s1):
  wid = _wid()
  sems = (s0, s1)

  def fetch(i, slot):
    base = i * C1
    sem = sems[slot]
    pltpu.sync_copy(pmap.at[pl.ds(base, C1)], pidx.at[slot])
    pltpu.sync_copy(qmap.at[pl.ds(base, C1)], qidx.at[slot])
    pltpu.async_copy(table.at[pl.ds(base, C1)], wrows.at[slot], sem)
    pltpu.async_copy(table.at[pidx.at[slot]], prows.at[slot], sem)
    pltpu.async_copy(table.at[qidx.at[slot]], qrows.at[slot], sem)

  def drain(i, slot):
    sem = sems[slot]
    pltpu.make_async_copy(table.at[pl.ds(0, C1)], wrows.at[slot], sem).wait()
    pltpu.make_async_copy(table.at[pl.ds(0, C1)], prows.at[slot], sem).wait()
    pltpu.make_async_copy(table.at[pl.ds(0, C1)], qrows.at[slot], sem).wait()

    def row(r, carry):
      for rr in range(2):
        for j in range(D // LANES):
          s = pl.ds(j * LANES, LANES)
          wrows[slot, 2 * r + rr, s] = (
              wrows[slot, 2 * r + rr, s]
              + prows[slot, 2 * r + rr, s]
              + qrows[slot, 2 * r + rr, s]
          )
      return carry

    lax.fori_loop(0, C1 // 2, row, 0)
    pltpu.sync_copy(wrows.at[slot], t2.at[pl.ds(i * C1, C1)])

  fetch(wid, 0)

  def body(m, carry):
    ia = wid + (2 * m) * NW
    ib = ia + NW
    ic = ib + NW

    @pl.when(ib < NCH1)
    def _():
      fetch(ib, 1)

    @pl.when(ia < NCH1)
    def _():
      drain(ia, 0)

    @pl.when(ic < NCH1)
    def _():
      fetch(ic, 0)

    @pl.when(ib < NCH1)
    def _():
      drain(ib, 1)

    return carry

  lax.fori_loop(0, ITER1 // 2, body, 0)


@functools.partial(
    pl.kernel,
    out_type=jax.ShapeDtypeStruct((B, L, D), jnp.float32),
    mesh=_MESH,
    compiler_params=_PARAMS,
    scratch_types=[
        pltpu.VMEM((2, L), jnp.int32),
        pltpu.VMEM((2, L, D), jnp.float32),
        pltpu.SemaphoreType.DMA,
        pltpu.SemaphoreType.DMA,
    ],
)
def _lookup(t2, seq, out, tok, rows, s0, s1):
  wid = _wid()
  b0 = wid * BPW
  sems = (s0, s1)

  def fetch(b, slot):
    sem = sems[slot]
    pltpu.sync_copy(seq.at[b], tok.at[slot])
    pltpu.async_copy(t2.at[tok.at[slot, pl.ds(0, LG0)]],
                     rows.at[slot, pl.ds(0, LG0)], sem)
    pltpu.async_copy(t2.at[tok.at[slot, pl.ds(LG0, LG1)]],
                     rows.at[slot, pl.ds(LG0, LG1)], sem)

  def drain(b, slot):
    sem = sems[slot]
    pltpu.make_async_copy(t2.at[pl.ds(0, LG0)],
                          rows.at[slot, pl.ds(0, LG0)], sem).wait()
    pltpu.make_async_copy(t2.at[pl.ds(0, LG1)],
                          rows.at[slot, pl.ds(LG0, LG1)], sem).wait()
    pltpu.sync_copy(rows.at[slot], out.at[b])

  fetch(b0, 0)

  def body(m, carry):
    b = b0 + 2 * m
    fetch(b + 1, 1)
    drain(b, 0)

    @pl.when(2 * m + 2 < BPW)
    def _():
      fetch(b + 2, 0)
    drain(b + 1, 1)
    return carry

  lax.fori_loop(0, BPW // 2, body, 0)


@jax.jit
def kernel(sequence, table, prefix_map, postfix_map):
  t2 = _build_t2(table, prefix_map, postfix_map)
  return _lookup(t2, sequence)
